# baseline (speedup 1.0000x reference)
import functools

import jax
import jax.numpy as jnp
from jax.experimental import pallas as pl
from jax.experimental.pallas import tpu as pltpu

B, S, H, R, E = 4, 2048, 2048, 256, 16
TOP_K = 2
TEMPERATURE = 1.0
LN_EPS = 1e-5

S_BLK = 512
N_BLK = S // S_BLK


def _router_kernel(x_ref, w1_ref, b1_ref, g_ref, bt_ref, w2_ref, b2_ref,
                   out_ref, acc_ref):
    i = pl.program_id(0)

    partial = jnp.sum(x_ref[...], axis=1)

    @pl.when(i == 0)
    def _init():
        acc_ref[...] = partial

    @pl.when(i > 0)
    def _acc():
        acc_ref[...] = acc_ref[...] + partial

    @pl.when(i == N_BLK - 1)
    def _finish():
        pooled = acc_ref[...] * (1.0 / S)
        h = jax.lax.dot_general(
            pooled, w1_ref[...], (((1,), (0,)), ((), ())),
            preferred_element_type=jnp.float32,
            precision=jax.lax.Precision.HIGHEST,
        ) + b1_ref[...]
        mu = jnp.mean(h, axis=-1, keepdims=True)
        var = jnp.mean((h - mu) ** 2, axis=-1, keepdims=True)
        h = (h - mu) * jax.lax.rsqrt(var + LN_EPS) * g_ref[...] + bt_ref[...]
        h = h * 0.5 * (1.0 + jax.lax.erf(h * (0.7071067811865476)))
        logits = jax.lax.dot_general(
            h, w2_ref[...], (((1,), (0,)), ((), ())),
            preferred_element_type=jnp.float32,
            precision=jax.lax.Precision.HIGHEST,
        ) + b2_ref[...]

        col = jax.lax.broadcasted_iota(jnp.int32, (B, E), 1)
        m1 = jnp.max(logits, axis=-1, keepdims=True)
        idx1 = jnp.min(jnp.where(logits == m1, col, E), axis=-1, keepdims=True)
        masked = jnp.where(col == idx1, -jnp.inf, logits)
        m2 = jnp.max(masked, axis=-1, keepdims=True)
        idx2 = jnp.min(jnp.where(masked == m2, col, E), axis=-1, keepdims=True)
        s = jnp.exp((m2 - m1) * (1.0 / TEMPERATURE))
        w_hi = 1.0 / (1.0 + s)
        w_lo = s / (1.0 + s)
        out_ref[...] = jnp.where(col == idx1, w_hi,
                                 jnp.where(col == idx2, w_lo, 0.0))


@functools.partial(jax.jit, static_argnames=())
def kernel(hidden_states, W1, b1, gamma, beta, W2, b2):
    b1r = b1.reshape(1, R)
    gr = gamma.reshape(1, R)
    btr = beta.reshape(1, R)
    b2r = b2.reshape(1, E)
    return pl.pallas_call(
        _router_kernel,
        grid=(N_BLK,),
        in_specs=[
            pl.BlockSpec((B, S_BLK, H), lambda i: (0, i, 0)),
            pl.BlockSpec((H, R), lambda i: (0, 0)),
            pl.BlockSpec((1, R), lambda i: (0, 0)),
            pl.BlockSpec((1, R), lambda i: (0, 0)),
            pl.BlockSpec((1, R), lambda i: (0, 0)),
            pl.BlockSpec((R, E), lambda i: (0, 0)),
            pl.BlockSpec((1, E), lambda i: (0, 0)),
        ],
        out_specs=pl.BlockSpec((B, E), lambda i: (0, 0)),
        out_shape=jax.ShapeDtypeStruct((B, E), jnp.float32),
        scratch_shapes=[pltpu.VMEM((B, H), jnp.float32)],
        compiler_params=pltpu.CompilerParams(
            dimension_semantics=("arbitrary",),
        ),
    )(hidden_states, W1, b1r, gr, btr, W2, b2r)


# speedup vs baseline: 1.0154x; 1.0154x over previous
import functools

import jax
import jax.numpy as jnp
from jax.experimental import pallas as pl
from jax.experimental.pallas import tpu as pltpu

B, S, H, R, E = 4, 2048, 2048, 256, 16
TOP_K = 2
TEMPERATURE = 1.0
LN_EPS = 1e-5

S_BLK = 128
N_BLK = S // S_BLK


def _router_kernel(x_ref, w1_ref, b1_ref, g_ref, bt_ref, w2_ref, b2_ref,
                   out_ref, acc_ref):
    i = pl.program_id(0)

    partial = jnp.sum(x_ref[...], axis=1)

    @pl.when(i == 0)
    def _init():
        acc_ref[...] = partial

    @pl.when(i > 0)
    def _acc():
        acc_ref[...] = acc_ref[...] + partial

    @pl.when(i == N_BLK - 1)
    def _finish():
        pooled = acc_ref[...] * (1.0 / S)
        h = jax.lax.dot_general(
            pooled, w1_ref[...], (((1,), (0,)), ((), ())),
            preferred_element_type=jnp.float32,
            precision=jax.lax.Precision.HIGHEST,
        ) + b1_ref[...]
        mu = jnp.mean(h, axis=-1, keepdims=True)
        var = jnp.mean((h - mu) ** 2, axis=-1, keepdims=True)
        h = (h - mu) * jax.lax.rsqrt(var + LN_EPS) * g_ref[...] + bt_ref[...]
        h = h * 0.5 * (1.0 + jax.lax.erf(h * (0.7071067811865476)))
        logits = jax.lax.dot_general(
            h, w2_ref[...], (((1,), (0,)), ((), ())),
            preferred_element_type=jnp.float32,
            precision=jax.lax.Precision.HIGHEST,
        ) + b2_ref[...]

        col = jax.lax.broadcasted_iota(jnp.int32, (B, E), 1)
        m1 = jnp.max(logits, axis=-1, keepdims=True)
        idx1 = jnp.min(jnp.where(logits == m1, col, E), axis=-1, keepdims=True)
        masked = jnp.where(col == idx1, -jnp.inf, logits)
        m2 = jnp.max(masked, axis=-1, keepdims=True)
        idx2 = jnp.min(jnp.where(masked == m2, col, E), axis=-1, keepdims=True)
        s = jnp.exp((m2 - m1) * (1.0 / TEMPERATURE))
        w_hi = 1.0 / (1.0 + s)
        w_lo = s / (1.0 + s)
        out_ref[...] = jnp.where(col == idx1, w_hi,
                                 jnp.where(col == idx2, w_lo, 0.0))


@functools.partial(jax.jit, static_argnames=())
def kernel(hidden_states, W1, b1, gamma, beta, W2, b2):
    b1r = b1.reshape(1, R)
    gr = gamma.reshape(1, R)
    btr = beta.reshape(1, R)
    b2r = b2.reshape(1, E)
    return pl.pallas_call(
        _router_kernel,
        grid=(N_BLK,),
        in_specs=[
            pl.BlockSpec((B, S_BLK, H), lambda i: (0, i, 0)),
            pl.BlockSpec((H, R), lambda i: (0, 0)),
            pl.BlockSpec((1, R), lambda i: (0, 0)),
            pl.BlockSpec((1, R), lambda i: (0, 0)),
            pl.BlockSpec((1, R), lambda i: (0, 0)),
            pl.BlockSpec((R, E), lambda i: (0, 0)),
            pl.BlockSpec((1, E), lambda i: (0, 0)),
        ],
        out_specs=pl.BlockSpec((B, E), lambda i: (0, 0)),
        out_shape=jax.ShapeDtypeStruct((B, E), jnp.float32),
        scratch_shapes=[pltpu.VMEM((B, H), jnp.float32)],
        compiler_params=pltpu.CompilerParams(
            dimension_semantics=("arbitrary",),
        ),
    )(hidden_states, W1, b1r, gr, btr, W2, b2r)
